# baseline (device time: 8290 ns/iter reference)
import jax
import jax.numpy as jnp
from jax import lax
from jax.experimental import pallas as pl
from jax.experimental.pallas import tpu as pltpu

N_DEV = 4
N_TOK = 256
D_IN = 128
D_OUT = 256
N_EXP = 8
E_LOCAL = 2
CAP = 25
ROWS = N_TOK // N_DEV


def kernel(x, router_W, route_idx, expert_W):
    del router_W

    def body(x_ref, idx_ref, ew_ref, out_ref, send_ref, recv_ref,
             x_vmem, ew_vmem, load_sems, send_sems, recv_sems):
        my = lax.axis_index("i")

        x_load = pltpu.make_async_copy(x_ref, x_vmem, load_sems.at[0])
        ew_load = pltpu.make_async_copy(ew_ref, ew_vmem, load_sems.at[1])
        x_load.start()
        ew_load.start()

        barrier = pltpu.get_barrier_semaphore()
        for k in range(1, N_DEV):
            peer = lax.rem(my + k, N_DEV)
            pl.semaphore_signal(barrier, inc=1, device_id=(peer,),
                                device_id_type=pl.DeviceIdType.MESH)

        idx = idx_ref[:, :]
        eids = lax.broadcasted_iota(jnp.int32, (N_TOK, N_EXP), 1)
        oh = (idx == eids).astype(jnp.bfloat16)
        ri = lax.broadcasted_iota(jnp.int32, (N_TOK, N_TOK), 0)
        ci = lax.broadcasted_iota(jnp.int32, (N_TOK, N_TOK), 1)
        tril = (ri >= ci).astype(jnp.bfloat16)
        ranks = jnp.dot(tril, oh, preferred_element_type=jnp.float32)
        keep = jnp.where(ranks <= float(CAP), oh.astype(jnp.float32), 0.0)

        masks = []
        for k in range(E_LOCAL):
            e = my * E_LOCAL + k
            m = jnp.sum(jnp.where(eids == e, keep, 0.0),
                        axis=1, keepdims=True)
            masks.append(m.astype(jnp.bfloat16))

        x_load.wait()
        ew_load.wait()
        ws = [ew_vmem[k, :, :].astype(jnp.bfloat16) for k in range(E_LOCAL)]
        xv = x_vmem[:, :].astype(jnp.bfloat16)

        a = jnp.concatenate([xv * masks[0], xv * masks[1]], axis=1)
        b = jnp.concatenate(ws, axis=0)
        part = jnp.dot(a, b, preferred_element_type=jnp.float32)

        for r in range(N_DEV):
            send_ref[r, :, :] = part[r * ROWS:(r + 1) * ROWS, :].astype(
                jnp.bfloat16)

        pl.semaphore_wait(barrier, N_DEV - 1)

        def send_to(peer):
            rdma = pltpu.make_async_remote_copy(
                src_ref=send_ref.at[peer],
                dst_ref=recv_ref.at[my],
                send_sem=send_sems.at[peer],
                recv_sem=recv_sems.at[my],
                device_id=(peer,),
                device_id_type=pl.DeviceIdType.MESH,
            )
            rdma.start()
            return rdma

        sends = [send_to(lax.rem(my + k, N_DEV)) for k in (2, 1, 3)]

        acc = send_ref[my, :, :]
        for k in (1, 2, 3):
            src = lax.rem(my + k, N_DEV)
            recv = pltpu.make_async_remote_copy(
                src_ref=send_ref.at[src],
                dst_ref=recv_ref.at[src],
                send_sem=send_sems.at[src],
                recv_sem=recv_sems.at[src],
                device_id=(src,),
                device_id_type=pl.DeviceIdType.MESH,
            )
            recv.wait_recv()
            acc = acc + recv_ref[src, :, :]

        out_ref[:, :] = acc

        for rdma in sends:
            rdma.wait_send()

    return pl.pallas_call(
        body,
        out_shape=jax.ShapeDtypeStruct((ROWS, D_OUT), jnp.bfloat16),
        in_specs=[
            pl.BlockSpec(memory_space=pl.ANY),
            pl.BlockSpec(memory_space=pltpu.VMEM),
            pl.BlockSpec(memory_space=pl.ANY),
        ],
        out_specs=pl.BlockSpec(memory_space=pltpu.VMEM),
        scratch_shapes=[
            pltpu.VMEM((N_DEV, ROWS, D_OUT), jnp.bfloat16),
            pltpu.VMEM((N_DEV, ROWS, D_OUT), jnp.bfloat16),
            pltpu.VMEM((N_TOK, D_IN), jnp.float32),
            pltpu.VMEM((E_LOCAL, D_IN, D_OUT), jnp.float32),
            pltpu.SemaphoreType.DMA((2,)),
            pltpu.SemaphoreType.DMA((N_DEV,)),
            pltpu.SemaphoreType.DMA((N_DEV,)),
        ],
        compiler_params=pltpu.CompilerParams(collective_id=0),
    )(x, route_idx, expert_W)


# device time: 8097 ns/iter; 1.0238x vs baseline; 1.0238x over previous
import jax
import jax.numpy as jnp
from jax import lax
from jax.experimental import pallas as pl
from jax.experimental.pallas import tpu as pltpu

N_DEV = 4
N_TOK = 256
D_IN = 128
D_OUT = 256
N_EXP = 8
E_LOCAL = 2
CAP = 25
ROWS = N_TOK // N_DEV


def kernel(x, router_W, route_idx, expert_W):
    del router_W

    def body(x_ref, idx_ref, ew_ref, out_ref, send_ref, recv_ref,
             send_sems, recv_sems):
        my = lax.axis_index("i")

        barrier = pltpu.get_barrier_semaphore()
        for k in range(1, N_DEV):
            peer = lax.rem(my + k, N_DEV)
            pl.semaphore_signal(barrier, inc=1, device_id=(peer,),
                                device_id_type=pl.DeviceIdType.MESH)

        idx = idx_ref[:, :]
        eids = lax.broadcasted_iota(jnp.int32, (N_TOK, N_EXP), 1)
        oh = (idx == eids).astype(jnp.bfloat16)
        ri = lax.broadcasted_iota(jnp.int32, (N_TOK, N_TOK), 0)
        ci = lax.broadcasted_iota(jnp.int32, (N_TOK, N_TOK), 1)
        tril = (ri >= ci).astype(jnp.bfloat16)
        ranks = jnp.dot(tril, oh, preferred_element_type=jnp.float32)
        keep = jnp.where(ranks <= float(CAP), oh.astype(jnp.float32), 0.0)

        masks = []
        for k in range(E_LOCAL):
            e = my * E_LOCAL + k
            m = jnp.sum(jnp.where(eids == e, keep, 0.0),
                        axis=1, keepdims=True)
            masks.append(m.astype(jnp.bfloat16))

        ws = [ew_ref[k, :, :].astype(jnp.bfloat16) for k in range(E_LOCAL)]
        xv = x_ref[:, :].astype(jnp.bfloat16)

        a = jnp.concatenate([xv * masks[0], xv * masks[1]], axis=1)
        b = jnp.concatenate(ws, axis=0)
        part = jnp.dot(a, b, preferred_element_type=jnp.float32)

        for r in range(N_DEV):
            send_ref[r, :, :] = part[r * ROWS:(r + 1) * ROWS, :].astype(
                jnp.bfloat16)

        pl.semaphore_wait(barrier, N_DEV - 1)

        def send_to(peer):
            rdma = pltpu.make_async_remote_copy(
                src_ref=send_ref.at[peer],
                dst_ref=recv_ref.at[my],
                send_sem=send_sems.at[peer],
                recv_sem=recv_sems.at[my],
                device_id=(peer,),
                device_id_type=pl.DeviceIdType.MESH,
            )
            rdma.start()
            return rdma

        sends = [send_to(lax.rem(my + k, N_DEV)) for k in (2, 1, 3)]

        acc = send_ref[my, :, :]
        for k in (1, 3, 2):
            src = lax.rem(my + k, N_DEV)
            recv = pltpu.make_async_remote_copy(
                src_ref=send_ref.at[src],
                dst_ref=recv_ref.at[src],
                send_sem=send_sems.at[src],
                recv_sem=recv_sems.at[src],
                device_id=(src,),
                device_id_type=pl.DeviceIdType.MESH,
            )
            recv.wait_recv()
            acc = acc + recv_ref[src, :, :]

        out_ref[:, :] = acc

        for rdma in sends:
            rdma.wait_send()

    return pl.pallas_call(
        body,
        out_shape=jax.ShapeDtypeStruct((ROWS, D_OUT), jnp.bfloat16),
        in_specs=[
            pl.BlockSpec(memory_space=pltpu.VMEM),
            pl.BlockSpec(memory_space=pltpu.VMEM),
            pl.BlockSpec(memory_space=pltpu.VMEM),
        ],
        out_specs=pl.BlockSpec(memory_space=pltpu.VMEM),
        scratch_shapes=[
            pltpu.VMEM((N_DEV, ROWS, D_OUT), jnp.bfloat16),
            pltpu.VMEM((N_DEV, ROWS, D_OUT), jnp.bfloat16),
            pltpu.SemaphoreType.DMA((N_DEV,)),
            pltpu.SemaphoreType.DMA((N_DEV,)),
        ],
        compiler_params=pltpu.CompilerParams(collective_id=0),
    )(x, route_idx, expert_W)
